# baseline (speedup 1.0000x reference)
import dataclasses
import functools

import jax
import jax.numpy as jnp
from jax import lax
from jax.experimental import pallas as pl
from jax.experimental.pallas import tpu as pltpu
from jax.experimental.pallas import tpu_sc as plsc

N = 10000
NP = 10240
E = 320000
NC = 2
NS = 16
NW = NC * NS
CH = 48
NCHUNK = 216
EPW = NCHUNK * CH
RPS = NP // NS
ZROWS = 32
NIB = 8
NRB = 4
F32 = jnp.float32

_SC_CP = pltpu.CompilerParams()
if "needs_layout_passes" in pltpu.CompilerParams.__dataclass_fields__:
    _SC_CP = dataclasses.replace(_SC_CP, needs_layout_passes=False,
                                 use_tc_tiling_on_sc=False)


def _splat(v16, j):
    idx = jnp.full((16, 1), j, jnp.int32)
    dn = lax.GatherDimensionNumbers(
        offset_dims=(), collapsed_slice_dims=(0,), start_index_map=(0,))
    return lax.gather(v16, idx, dn, slice_sizes=(1,),
                      mode=lax.GatherScatterMode.PROMISE_IN_BOUNDS)


def _segsum(parts, packed):
    nparts = len(parts)
    mesh = plsc.VectorSubcoreMesh(core_axis_name="c", subcore_axis_name="s")
    out_type = jax.ShapeDtypeStruct((NC, nparts, NP, 128), F32)
    scratch = (
        [pltpu.VMEM((3, CH), jnp.int32) for _ in range(NIB)]
        + [pltpu.VMEM((CH, 64), jnp.int32) for _ in range(NRB)]
        + [pltpu.VMEM((CH, 128), F32) for _ in range(NRB)]
        + [pltpu.VMEM((ZROWS, 128), F32),
           pltpu.VMEM_SHARED((NP, 128), F32)]
        + [pltpu.SemaphoreType.DMA for _ in range(NIB + 2 * NRB)]
    )

    @functools.partial(pl.kernel, out_type=out_type, mesh=mesh,
                       scratch_types=scratch, compiler_params=_SC_CP)
    def k(*refs):
        part_h = refs[:nparts]
        packed_h, out_h = refs[nparts], refs[nparts + 1]
        rest = refs[nparts + 2:]
        idx_v = rest[:NIB]
        rows_v = rest[NIB:NIB + NRB]
        stage_v = rest[NIB + NRB:NIB + 2 * NRB]
        zbuf = rest[NIB + 2 * NRB]
        acc = rest[NIB + 2 * NRB + 1]
        sems = rest[NIB + 2 * NRB + 2:]
        sem_i = sems[:NIB]
        sem_g = sems[NIB:NIB + NRB]
        sem_s = sems[NIB + NRB:]

        ci = lax.axis_index("c")
        s = lax.axis_index("s")
        chunk0 = (s * NC + ci) * NCHUNK

        def idx_cp(slot, lin):
            return pltpu.make_async_copy(packed_h.at[lin], idx_v[slot],
                                         sem_i[slot])

        def gath_cp(p, slot, rb):
            return pltpu.make_async_copy(part_h[p].at[idx_v[slot].at[0]],
                                         rows_v[rb], sem_g[rb])

        def scat_cp(rb, slot):
            return pltpu.make_async_copy(stage_v[rb],
                                         acc.at[idx_v[slot].at[1]],
                                         sem_s[rb])

        def scale(rb, slot):
            @pl.loop(0, CH // 16)
            def _(g):
                wi = idx_v[slot][2, pl.ds(g * 16, 16)]
                w16 = plsc.bitcast(wi, F32)
                for j in range(16):
                    wj = _splat(w16, j)
                    e = g * 16 + j
                    for cc in range(4):
                        v = rows_v[rb][e, pl.ds(cc * 16, 16)]
                        a, b2 = plsc.unpack(
                            plsc.bitcast(v, jnp.bfloat16),
                            format=plsc.PackFormat.INTERLEAVED)
                        stage_v[rb].at[e, pl.ds(cc * 32, 16)][...] = a * wj
                        stage_v[rb].at[e, pl.ds(cc * 32 + 16, 16)][...] = \
                            b2 * wj

        zero = jnp.zeros((16,), F32)

        @pl.loop(0, ZROWS)
        def _(r):
            for cc in range(8):
                zbuf.at[r, pl.ds(cc * 16, 16)][...] = zero

        for p in range(nparts):
            for blk in range(RPS // ZROWS):
                pltpu.sync_copy(zbuf, acc.at[pl.ds(s * RPS + blk * ZROWS,
                                                   ZROWS)])
            plsc.subcore_barrier()

            for b in range(NRB):
                idx_cp(b, chunk0 + b).start()

            @pl.loop(0, NCHUNK, step=NIB)
            def _(kk):
                for b in range(NIB):
                    c = kk + b
                    rb = b % NRB

                    @pl.when(c >= NRB)
                    def _():
                        scat_cp(rb, (b + NRB) % NIB).wait()

                    idx_cp(b, chunk0 + c).wait()
                    gath_cp(p, b, rb).start()

                    @pl.when(c >= 1)
                    def _():
                        pb = (b - 1) % NIB
                        pr = (b - 1) % NRB
                        gath_cp(p, pb, pr).wait()
                        scale(pr, pb)
                        scat_cp(pr, pb).start(add=True)

                    @pl.when(c + NRB < NCHUNK)
                    def _():
                        ns = (b + NRB) % NIB
                        idx_cp(ns, chunk0 + c + NRB).start()

            lb = (NCHUNK - 1) % NIB
            lr = (NCHUNK - 1) % NRB
            gath_cp(p, lb, lr).wait()
            scale(lr, lb)
            scat_cp(lr, lb).start(add=True)
            for rb in range(NRB):
                scat_cp(rb, (rb + NRB) % NIB).wait()

            plsc.subcore_barrier()
            pltpu.sync_copy(acc.at[pl.ds(s * RPS, RPS)],
                            out_h.at[ci, p, pl.ds(s * RPS, RPS)])
            plsc.subcore_barrier()

    return k(*parts, packed)


def _perm_pairs(z, d):
    n = z.shape[0]
    return z.reshape(n, d // 32, 2, 16).swapaxes(2, 3).reshape(n, d)


def _gnn_layer(P, hs, W_rel, W_root, b, emit_bf=True):
    nparts = P.shape[1]
    BN = 1000
    grid = (N // BN,)
    in_specs = [pl.BlockSpec((NC, nparts, BN, 128), lambda i: (0, 0, i, 0))]
    in_specs += [pl.BlockSpec((BN, 128), lambda i: (i, 0)) for _ in hs]
    in_specs += [
        pl.BlockSpec(W_rel.shape, lambda i: (0, 0)),
        pl.BlockSpec(W_root.shape, lambda i: (0, 0)),
        pl.BlockSpec((1, 256), lambda i: (0, 0)),
    ]
    nout = 4 if emit_bf else 2
    out_specs = [pl.BlockSpec((BN, 128), lambda i: (i, 0))] * nout
    nh = len(hs)

    def body(P_ref, *refs):
        h_refs = refs[:nh]
        wrel, wroot, b_ref = refs[nh:nh + 3]
        outs = refs[nh + 3:]
        olo, ohi = outs[0], outs[1]
        acc = jnp.zeros((BN, 256), F32)
        for p in range(nparts):
            aggp = P_ref[0, p] + P_ref[1, p]
            acc += jnp.dot(aggp, wrel[p * 128:(p + 1) * 128],
                           preferred_element_type=F32)
        for q in range(nh):
            acc += jnp.dot(h_refs[q][...], wroot[q * 128:(q + 1) * 128],
                           preferred_element_type=F32)
        z = jnp.maximum(acc + b_ref[...], 0.0)
        olo[...] = z[:, :128]
        ohi[...] = z[:, 128:]
        if emit_bf:
            oblo, obhi = outs[2:]
            zp = _perm_pairs(z, 256).astype(jnp.bfloat16)
            oblo[...] = zp[:, :128]
            obhi[...] = zp[:, 128:]

    out_shape = [jax.ShapeDtypeStruct((N, 128), F32)] * 2
    if emit_bf:
        out_shape += [jax.ShapeDtypeStruct((N, 128), jnp.bfloat16)] * 2
    return pl.pallas_call(
        body, grid=grid, in_specs=in_specs, out_specs=out_specs,
        out_shape=out_shape,
    )(P, *hs, W_rel, W_root, b.reshape(1, -1))


def _x_to_bf16(x):
    BN = 1000
    grid = (N // BN,)

    def body(xr, o):
        o[...] = _perm_pairs(xr[...], 128).astype(jnp.bfloat16)

    return pl.pallas_call(
        body, grid=grid,
        in_specs=[pl.BlockSpec((BN, 128), lambda i: (i, 0))],
        out_specs=pl.BlockSpec((BN, 128), lambda i: (i, 0)),
        out_shape=jax.ShapeDtypeStruct((N, 128), jnp.bfloat16),
    )(x)


def _mlp_head(h_lo, h_hi, Wfc, bfc, Wlast, blast):
    BN = 1000
    grid = (N // BN,)
    in_specs = [
        pl.BlockSpec((BN, 128), lambda i: (i, 0)),
        pl.BlockSpec((BN, 128), lambda i: (i, 0)),
        pl.BlockSpec(Wfc.shape, lambda i: (0, 0)),
        pl.BlockSpec((1, 256), lambda i: (0, 0)),
        pl.BlockSpec(Wlast.shape, lambda i: (0, 0)),
        pl.BlockSpec((1, Wlast.shape[1]), lambda i: (0, 0)),
    ]
    out_specs = pl.BlockSpec((BN, Wlast.shape[1]), lambda i: (i, 0))

    def body(hlo, hhi, wfc, bfc_r, wlast, blast_r, o):
        t = (jnp.dot(hlo[...], wfc[:128], preferred_element_type=F32)
             + jnp.dot(hhi[...], wfc[128:], preferred_element_type=F32)
             + bfc_r[...])
        t = jnp.maximum(t, 0.0)
        t = jnp.maximum(
            jnp.dot(t, wfc[...], preferred_element_type=F32) + bfc_r[...],
            0.0)
        o[...] = jnp.dot(t, wlast[...], preferred_element_type=F32) \
            + blast_r[...]

    return pl.pallas_call(
        body, grid=grid, in_specs=in_specs, out_specs=out_specs,
        out_shape=jax.ShapeDtypeStruct((N, Wlast.shape[1]), F32),
    )(h_lo, h_hi, Wfc, bfc.reshape(1, -1), Wlast, blast.reshape(1, -1))


def kernel(x, edge_index, edge_attr, W1_rel, W1_root, b1,
           W2_rel, W2_root, b2, Wfc, bfc, Wlast, blast):
    epw_real = E // NW
    wbits = lax.bitcast_convert_type(edge_attr, jnp.int32)
    packed = jnp.stack([edge_index[0], edge_index[1], wbits])
    packed = packed.reshape(3, NW, epw_real)
    packed = jnp.pad(packed, ((0, 0), (0, 0), (0, EPW - epw_real)))
    packed = packed.reshape(3, NW, NCHUNK, CH).transpose(1, 2, 0, 3)
    packed = packed.reshape(NW * NCHUNK, 3, CH)

    def _pk(hb):
        return lax.bitcast_convert_type(hb.reshape(N, 64, 2), jnp.int32)

    xb = _x_to_bf16(x)
    P1 = _segsum([_pk(xb)], packed)
    h1_lo, h1_hi, h1_lob, h1_hib = _gnn_layer(P1, [x], W1_rel, W1_root, b1)

    P2 = _segsum([_pk(h1_lob), _pk(h1_hib)], packed)
    h2_lo, h2_hi, h2_lob, h2_hib = _gnn_layer(P2, [h1_lo, h1_hi],
                                              W2_rel, W2_root, b2)

    P3 = _segsum([_pk(h2_lob), _pk(h2_hib)], packed)
    h3_lo, h3_hi = _gnn_layer(P3, [h2_lo, h2_hi], W2_rel, W2_root, b2,
                              emit_bf=False)

    return _mlp_head(h3_lo, h3_hi, Wfc, bfc, Wlast, blast)


# speedup vs baseline: 1.7103x; 1.7103x over previous
import functools

import jax
import jax.numpy as jnp
from jax import lax
from jax.experimental import pallas as pl
from jax.experimental.pallas import tpu as pltpu
from jax.experimental.pallas import tpu_sc as plsc

N = 10000
NP = 10240
E = 320000
NC = 2
NS = 16
NW = NC * NS
EPW = E // NW
CH = 80
NCHUNK = EPW // CH
RPS = NP // NS
ZROWS = 128
F32 = jnp.float32


def _splat(v16, j):
    idx = jnp.full((16, 1), j, jnp.int32)
    dn = lax.GatherDimensionNumbers(
        offset_dims=(), collapsed_slice_dims=(0,), start_index_map=(0,))
    return lax.gather(v16, idx, dn, slice_sizes=(1,),
                      mode=lax.GatherScatterMode.PROMISE_IN_BOUNDS)


def _segsum(parts, src, dst, w):
    nparts = len(parts)
    mesh = plsc.VectorSubcoreMesh(core_axis_name="c", subcore_axis_name="s")
    out_type = jax.ShapeDtypeStruct((NC, nparts, NP, 128), F32)
    scratch = [
        pltpu.VMEM((CH,), jnp.int32),
        pltpu.VMEM((CH,), jnp.int32),
        pltpu.VMEM((CH,), F32),
        pltpu.VMEM((CH, 128), F32),
        pltpu.VMEM((ZROWS, 128), F32),
        pltpu.VMEM_SHARED((NP, 128), F32),
        pltpu.SemaphoreType.DMA,
    ]

    @functools.partial(pl.kernel, out_type=out_type, mesh=mesh,
                       scratch_types=scratch)
    def k(*refs):
        part_h = refs[:nparts]
        (src_h, dst_h, w_h, out_h,
         src_v, dst_v, w_v, rows_v, zbuf, acc, sem) = refs[nparts:]
        c = lax.axis_index("c")
        s = lax.axis_index("s")
        base0 = (s * NC + c) * EPW

        zero = jnp.zeros((16,), F32)

        @pl.loop(0, ZROWS)
        def _(r):
            for cc in range(8):
                zbuf.at[r, pl.ds(cc * 16, 16)][...] = zero

        for p in range(nparts):
            for blk in range(RPS // ZROWS):
                pltpu.sync_copy(zbuf, acc.at[pl.ds(s * RPS + blk * ZROWS,
                                                   ZROWS)])
            plsc.subcore_barrier()

            @pl.loop(0, NCHUNK)
            def _(kk):
                base = base0 + kk * CH
                pltpu.sync_copy(src_h.at[pl.ds(base, CH)], src_v)
                pltpu.sync_copy(dst_h.at[pl.ds(base, CH)], dst_v)
                pltpu.sync_copy(w_h.at[pl.ds(base, CH)], w_v)
                pltpu.async_copy(part_h[p].at[src_v], rows_v, sem).wait()

                @pl.loop(0, CH // 16)
                def _(g):
                    w16 = w_v[pl.ds(g * 16, 16)]
                    for j in range(16):
                        wj = _splat(w16, j)
                        for cc in range(8):
                            sl = (g * 16 + j, pl.ds(cc * 16, 16))
                            rows_v.at[sl][...] = rows_v.at[sl][...] * wj

                pltpu.sync_copy(rows_v, acc.at[dst_v], add=True)

            plsc.subcore_barrier()
            pltpu.sync_copy(acc.at[pl.ds(s * RPS, RPS)],
                            out_h.at[c, p, pl.ds(s * RPS, RPS)])
            plsc.subcore_barrier()

    return k(*parts, src, dst, w)


def _gnn_layer(P, hs, W_rel, W_root, b):
    nparts = P.shape[1]
    BN = 1000
    grid = (N // BN,)
    in_specs = [pl.BlockSpec((NC, nparts, BN, 128), lambda i: (0, 0, i, 0))]
    in_specs += [pl.BlockSpec((BN, 128), lambda i: (i, 0)) for _ in hs]
    in_specs += [
        pl.BlockSpec(W_rel.shape, lambda i: (0, 0)),
        pl.BlockSpec(W_root.shape, lambda i: (0, 0)),
        pl.BlockSpec((1, 256), lambda i: (0, 0)),
    ]
    out_specs = [pl.BlockSpec((BN, 128), lambda i: (i, 0))] * 2
    nh = len(hs)

    def body(P_ref, *refs):
        h_refs = refs[:nh]
        wrel, wroot, b_ref, olo, ohi = refs[nh:]
        acc = jnp.zeros((BN, 256), F32)
        for p in range(nparts):
            aggp = P_ref[0, p] + P_ref[1, p]
            acc += jnp.dot(aggp, wrel[p * 128:(p + 1) * 128],
                           preferred_element_type=F32)
        for q in range(nh):
            acc += jnp.dot(h_refs[q][...], wroot[q * 128:(q + 1) * 128],
                           preferred_element_type=F32)
        z = jnp.maximum(acc + b_ref[...], 0.0)
        olo[...] = z[:, :128]
        ohi[...] = z[:, 128:]

    return pl.pallas_call(
        body, grid=grid, in_specs=in_specs, out_specs=out_specs,
        out_shape=[jax.ShapeDtypeStruct((N, 128), F32)] * 2,
    )(P, *hs, W_rel, W_root, b.reshape(1, -1))


def _mlp_head(h_lo, h_hi, Wfc, bfc, Wlast, blast):
    BN = 1000
    grid = (N // BN,)
    in_specs = [
        pl.BlockSpec((BN, 128), lambda i: (i, 0)),
        pl.BlockSpec((BN, 128), lambda i: (i, 0)),
        pl.BlockSpec(Wfc.shape, lambda i: (0, 0)),
        pl.BlockSpec((1, 256), lambda i: (0, 0)),
        pl.BlockSpec(Wlast.shape, lambda i: (0, 0)),
        pl.BlockSpec((1, Wlast.shape[1]), lambda i: (0, 0)),
    ]
    out_specs = pl.BlockSpec((BN, Wlast.shape[1]), lambda i: (i, 0))

    def body(hlo, hhi, wfc, bfc_r, wlast, blast_r, o):
        t = (jnp.dot(hlo[...], wfc[:128], preferred_element_type=F32)
             + jnp.dot(hhi[...], wfc[128:], preferred_element_type=F32)
             + bfc_r[...])
        t = jnp.maximum(t, 0.0)
        t = jnp.maximum(
            jnp.dot(t, wfc[...], preferred_element_type=F32) + bfc_r[...],
            0.0)
        o[...] = jnp.dot(t, wlast[...], preferred_element_type=F32) \
            + blast_r[...]

    return pl.pallas_call(
        body, grid=grid, in_specs=in_specs, out_specs=out_specs,
        out_shape=jax.ShapeDtypeStruct((N, Wlast.shape[1]), F32),
    )(h_lo, h_hi, Wfc, bfc.reshape(1, -1), Wlast, blast.reshape(1, -1))


def kernel(x, edge_index, edge_attr, W1_rel, W1_root, b1,
           W2_rel, W2_root, b2, Wfc, bfc, Wlast, blast):
    src = edge_index[0]
    dst = edge_index[1]

    P1 = _segsum([x], src, dst, edge_attr)
    h1_lo, h1_hi = _gnn_layer(P1, [x], W1_rel, W1_root, b1)

    P2 = _segsum([h1_lo, h1_hi], src, dst, edge_attr)
    h2_lo, h2_hi = _gnn_layer(P2, [h1_lo, h1_hi], W2_rel, W2_root, b2)

    P3 = _segsum([h2_lo, h2_hi], src, dst, edge_attr)
    h3_lo, h3_hi = _gnn_layer(P3, [h2_lo, h2_hi], W2_rel, W2_root, b2)

    return _mlp_head(h3_lo, h3_hi, Wfc, bfc, Wlast, blast)
